# baseline (speedup 1.0000x reference)
import dataclasses
import functools

import jax
import jax.numpy as jnp
from jax import lax
from jax.experimental import pallas as pl
from jax.experimental.pallas import tpu as pltpu
from jax.experimental.pallas import tpu_sc as plsc

NC = 2
NS = 16
L = 16
NW = NC * NS
D = 128
DC = D // L
NBUF = 3


def _make_sc_scores(B, b_per_w, chunk):
    mesh = plsc.VectorSubcoreMesh(core_axis_name="c", subcore_axis_name="s")
    nchunks = b_per_w // chunk

    cp = pltpu.CompilerParams()
    if "needs_layout_passes" in pltpu.CompilerParams.__dataclass_fields__:
        cp = dataclasses.replace(cp, needs_layout_passes=False)

    nbuf = min(NBUF, nchunks)
    row_scratch = [pltpu.VMEM((chunk, D), jnp.float32) for _ in range(2 * nbuf)]
    sem_scratch = [pltpu.SemaphoreType.DMA for _ in range(2 * nbuf)]

    @functools.partial(
        pl.kernel,
        mesh=mesh,
        compiler_params=cp,
        out_type=jax.ShapeDtypeStruct((B,), jnp.float32),
        scratch_types=[
            pltpu.VMEM((b_per_w,), jnp.int32),
            pltpu.VMEM((b_per_w,), jnp.int32),
            pltpu.VMEM((b_per_w, L), jnp.float32),
            pltpu.VMEM((b_per_w,), jnp.float32),
            pltpu.VMEM((2 * D,), jnp.float32),
            pltpu.VMEM((1,), jnp.float32),
            pltpu.SemaphoreType.DMA,
            pltpu.SemaphoreType.DMA,
            pltpu.SemaphoreType.DMA,
            pltpu.SemaphoreType.DMA,
        ] + row_scratch + sem_scratch,
    )
    def sc_scores(uidx_hbm, sidx_hbm, ut_hbm, st_hbm, w_hbm, b_hbm, out_hbm,
                  uidx_v, sidx_v, part_v, out_v, w_v, b_v, sem_w, sem_b,
                  sem_iu, sem_is, *bufs_and_sems):
        u_bufs = bufs_and_sems[0:nbuf]
        s_bufs = bufs_and_sems[nbuf:2 * nbuf]
        sem_u = bufs_and_sems[2 * nbuf:3 * nbuf]
        sem_s = bufs_and_sems[3 * nbuf:4 * nbuf]

        wid = lax.axis_index("s") * NC + lax.axis_index("c")
        base = wid * b_per_w
        cw = pltpu.async_copy(w_hbm.at[0], w_v, sem_w)
        cb = pltpu.async_copy(b_hbm, b_v, sem_b)
        ciu = pltpu.async_copy(uidx_hbm.at[pl.ds(base, b_per_w)], uidx_v, sem_iu)
        cis = pltpu.async_copy(sidx_hbm.at[pl.ds(base, b_per_w)], sidx_v, sem_is)
        ciu.wait()
        cis.wait()

        def issue(c):
            b = c % nbuf
            cu = pltpu.async_copy(
                ut_hbm.at[uidx_v.at[pl.ds(c * chunk, chunk)]],
                u_bufs[b], sem_u[b])
            cs = pltpu.async_copy(
                st_hbm.at[sidx_v.at[pl.ds(c * chunk, chunk)]],
                s_bufs[b], sem_s[b])
            return cu, cs

        handles = [None] * nchunks
        for c in range(nbuf):
            handles[c] = issue(c)

        cw.wait()
        cb.wait()
        wv = [w_v[pl.ds(k * L, L)] for k in range(2 * DC)]
        bvec = plsc.load_gather(b_v, [jnp.full((L,), 0, jnp.int32)])
        for c in range(nchunks):
            cu, cs = handles[c]
            cu.wait()
            cs.wait()
            ub, sb = u_bufs[c % nbuf], s_bufs[c % nbuf]

            @pl.loop(0, chunk)
            def _(i, c=c, ub=ub, sb=sb):
                acc = ub[i, pl.ds(0, L)] * wv[0]
                for k in range(1, DC):
                    acc = acc + ub[i, pl.ds(k * L, L)] * wv[k]
                for k in range(DC):
                    acc = acc + sb[i, pl.ds(k * L, L)] * wv[DC + k]
                part_v[c * chunk + i, :] = acc

            if c + nbuf < nchunks:
                handles[c + nbuf] = issue(c + nbuf)

            @pl.loop(c * chunk, (c + 1) * chunk, step=L)
            def _(i0):
                rows = i0 + lax.iota(jnp.int32, L)
                tot = plsc.load_gather(
                    part_v, [rows, jnp.full((L,), 0, jnp.int32)])
                for j in range(1, L):
                    tot = tot + plsc.load_gather(
                        part_v, [rows, jnp.full((L,), j, jnp.int32)])
                out_v[pl.ds(i0, L)] = tot + bvec

        pltpu.sync_copy(out_v, out_hbm.at[pl.ds(base, b_per_w)])

    return sc_scores


def kernel(user, skill, user_table, skill_table, fc_w, fc_b):
    B = user.shape[0]
    b_per_w = B // NW
    chunk = min(b_per_w, 64)
    uidx = user.astype(jnp.int32)
    sidx = skill.astype(jnp.int32)
    return _make_sc_scores(B, b_per_w, chunk)(
        uidx, sidx, user_table, skill_table, fc_w, fc_b)


# speedup vs baseline: 1.0216x; 1.0216x over previous
import dataclasses
import functools

import jax
import jax.numpy as jnp
from jax import lax
from jax.experimental import pallas as pl
from jax.experimental.pallas import tpu as pltpu
from jax.experimental.pallas import tpu_sc as plsc

NC = 2
NS = 16
L = 16
NW = NC * NS
D = 128
DC = D // L
NBUF = 3


def _make_sc_scores(B, b_per_w, chunk):
    mesh = plsc.VectorSubcoreMesh(core_axis_name="c", subcore_axis_name="s")
    nchunks = b_per_w // chunk

    cp = pltpu.CompilerParams()
    if "needs_layout_passes" in pltpu.CompilerParams.__dataclass_fields__:
        cp = dataclasses.replace(cp, needs_layout_passes=False)

    nbuf = min(NBUF, nchunks)
    row_scratch = [pltpu.VMEM((chunk, D), jnp.float32) for _ in range(2 * nbuf)]
    sem_scratch = [pltpu.SemaphoreType.DMA for _ in range(2 * nbuf)]

    @functools.partial(
        pl.kernel,
        mesh=mesh,
        compiler_params=cp,
        out_type=jax.ShapeDtypeStruct((B,), jnp.float32),
        scratch_types=[
            pltpu.VMEM((b_per_w,), jnp.int32),
            pltpu.VMEM((b_per_w,), jnp.int32),
            pltpu.VMEM((b_per_w, L), jnp.float32),
            pltpu.VMEM((b_per_w,), jnp.float32),
            pltpu.VMEM((2 * D,), jnp.float32),
            pltpu.VMEM((1,), jnp.float32),
            pltpu.SemaphoreType.DMA,
            pltpu.SemaphoreType.DMA,
            pltpu.SemaphoreType.DMA,
            pltpu.SemaphoreType.DMA,
        ] + row_scratch + sem_scratch,
    )
    def sc_scores(uidx_hbm, sidx_hbm, ut_hbm, st_hbm, w_hbm, b_hbm, out_hbm,
                  uidx_v, sidx_v, part_v, out_v, w_v, b_v, sem_w, sem_b,
                  sem_iu, sem_is, *bufs_and_sems):
        u_bufs = bufs_and_sems[0:nbuf]
        s_bufs = bufs_and_sems[nbuf:2 * nbuf]
        sem_u = bufs_and_sems[2 * nbuf:3 * nbuf]
        sem_s = bufs_and_sems[3 * nbuf:4 * nbuf]

        wid = lax.axis_index("s") * NC + lax.axis_index("c")
        base = wid * b_per_w
        cw = pltpu.async_copy(w_hbm.at[0], w_v, sem_w)
        cb = pltpu.async_copy(b_hbm, b_v, sem_b)
        ciu = pltpu.async_copy(uidx_hbm.at[pl.ds(base, b_per_w)], uidx_v, sem_iu)
        cis = pltpu.async_copy(sidx_hbm.at[pl.ds(base, b_per_w)], sidx_v, sem_is)
        ciu.wait()
        cis.wait()

        def issue(c):
            b = c % nbuf
            cu = pltpu.async_copy(
                ut_hbm.at[uidx_v.at[pl.ds(c * chunk, chunk)]],
                u_bufs[b], sem_u[b])
            cs = pltpu.async_copy(
                st_hbm.at[sidx_v.at[pl.ds(c * chunk, chunk)]],
                s_bufs[b], sem_s[b])
            return cu, cs

        handles = [None] * nchunks
        for c in range(nbuf):
            handles[c] = issue(c)

        cw.wait()
        cb.wait()
        wv = [w_v[pl.ds(k * L, L)] for k in range(2 * DC)]
        bvec = plsc.load_gather(b_v, [jnp.full((L,), 0, jnp.int32)])
        for c in range(nchunks):
            cu, cs = handles[c]
            cu.wait()
            cs.wait()
            ub, sb = u_bufs[c % nbuf], s_bufs[c % nbuf]

            @pl.loop(0, chunk)
            def _(i, c=c, ub=ub, sb=sb):
                acc = ub[i, pl.ds(0, L)] * wv[0]
                for k in range(1, DC):
                    acc = acc + ub[i, pl.ds(k * L, L)] * wv[k]
                for k in range(DC):
                    acc = acc + sb[i, pl.ds(k * L, L)] * wv[DC + k]
                part_v[c * chunk + i, :] = acc

            if c + nbuf < nchunks:
                handles[c + nbuf] = issue(c + nbuf)

        @pl.loop(0, b_per_w, step=L)
        def _(i0):
            rows = i0 + lax.iota(jnp.int32, L)
            tot = plsc.load_gather(part_v, [rows, jnp.full((L,), 0, jnp.int32)])
            for j in range(1, L):
                tot = tot + plsc.load_gather(
                    part_v, [rows, jnp.full((L,), j, jnp.int32)])
            out_v[pl.ds(i0, L)] = tot + bvec

        pltpu.sync_copy(out_v, out_hbm.at[pl.ds(base, b_per_w)])

    return sc_scores


def kernel(user, skill, user_table, skill_table, fc_w, fc_b):
    B = user.shape[0]
    b_per_w = B // NW
    chunk = min(b_per_w, 64)
    uidx = user.astype(jnp.int32)
    sidx = skill.astype(jnp.int32)
    return _make_sc_scores(B, b_per_w, chunk)(
        uidx, sidx, user_table, skill_table, fc_w, fc_b)
